# baseline (speedup 1.0000x reference)
import functools

import jax
import jax.numpy as jnp
from jax import lax
from jax.experimental import pallas as pl
from jax.experimental.pallas import tpu as pltpu
from jax.experimental.pallas import tpu_sc as plsc

N = 10000
E = 40000
E2 = 120000
NUM_REL = 7
NUM_ANGLE = 8
NUM_GRAPHS = 32
EPS = 1e-5
BLK = 512

_NC = 2
_NS = 16
_NW = _NC * _NS
_SUB = 64
_SSUB = 128
_CROWS = 2048


def _round_up(x, m):
    return ((x + m - 1) // m) * m


def _sc_gather(table, idx):
    t, d = table.shape
    p = idx.shape[0]
    assert p % (_NW * _SUB) == 0 and (d * 4) % 64 == 0
    per_w = p // _NW
    nsub = per_w // _SUB
    mesh = plsc.VectorSubcoreMesh(core_axis_name="c", subcore_axis_name="s")

    def body(table_hbm, idx_hbm, out_hbm, idx_v, buf0, buf1, buf2,
             g0, g1, g2, s0, s1, s2):
        wid = lax.axis_index("s") * _NC + lax.axis_index("c")
        base = wid * per_w
        pltpu.sync_copy(idx_hbm.at[pl.ds(base, per_w)], idx_v)
        bufs = (buf0, buf1, buf2)
        gsem = (g0, g1, g2)
        ssem = (s0, s1, s2)
        gath = [None, None, None]
        stor = [None, None, None]

        def start_gather(j):
            s = j % 3
            gath[s] = pltpu.async_copy(
                table_hbm.at[idx_v.at[pl.ds(j * _SUB, _SUB)]],
                bufs[s], gsem[s])

        for j in range(min(2, nsub)):
            start_gather(j)
        for j in range(nsub):
            s = j % 3
            sn = (j + 2) % 3
            if j + 2 < nsub:
                if stor[sn] is not None:
                    stor[sn].wait()
                    stor[sn] = None
                start_gather(j + 2)
            gath[s].wait()
            if stor[s] is not None:
                stor[s].wait()
            stor[s] = pltpu.async_copy(
                bufs[s], out_hbm.at[pl.ds(base + j * _SUB, _SUB)], ssem[s])
        for s in range(3):
            if stor[s] is not None:
                stor[s].wait()

    f = pl.kernel(
        body,
        out_type=jax.ShapeDtypeStruct((p, d), jnp.float32),
        mesh=mesh,
        scratch_types=[
            pltpu.VMEM((per_w,), jnp.int32),
            pltpu.VMEM((_SUB, d), jnp.float32),
            pltpu.VMEM((_SUB, d), jnp.float32),
            pltpu.VMEM((_SUB, d), jnp.float32),
            pltpu.SemaphoreType.DMA,
            pltpu.SemaphoreType.DMA,
            pltpu.SemaphoreType.DMA,
            pltpu.SemaphoreType.DMA,
            pltpu.SemaphoreType.DMA,
            pltpu.SemaphoreType.DMA,
        ],
    )
    return f(table, idx)


def _sc_scatter(vals, order, sdst, tbounds, m_pad):
    p, d = vals.shape
    assert (d * 4) % 64 == 0 and p % 8 == 0 and m_pad % (_NW * 8) == 0
    rows_pt = m_pad // _NW
    nvec = d // 16
    zrows = 32
    oslots = 64
    mesh = plsc.VectorSubcoreMesh(core_axis_name="c", subcore_axis_name="s")

    def _read_i32(ref, idx):
        return ref[pl.ds(idx, 16)][0]

    def body(vals_hbm, order_hbm, sdst_hbm, tb_hbm, zeros_hbm, out_hbm,
             tb_v, ord_v, sdst_v, dlist_v, gbuf, obuf, zbuf, abuf, sem):
        cid = lax.axis_index("c")
        sid = lax.axis_index("s")
        wid = sid * _NC + cid
        pltpu.sync_copy(tb_hbm, tb_v)
        pltpu.sync_copy(zeros_hbm, zbuf)
        obase = wid * rows_pt
        for h in range(rows_pt // zrows):
            pltpu.sync_copy(zbuf, out_hbm.at[pl.ds(obase + h * zrows, zrows)])
        lo = _read_i32(tb_v, wid)
        hi = _read_i32(tb_v, wid + 1)
        lo8 = (lo // 8) * 8
        nb = (hi - lo8 + _SSUB - 1) // _SSUB

        def flush_obuf():
            pltpu.sync_copy(obuf, out_hbm.at[dlist_v])

        def reset_dlist():
            for k in range(oslots // 16):
                dlist_v[pl.ds(k * 16, 16)] = jnp.broadcast_to(
                    jnp.int32(m_pad), (16,))

        def set_dlist(slot, val):
            base = (slot // 16) * 16
            lane = slot % 16
            v = dlist_v[pl.ds(base, 16)]
            dlist_v[pl.ds(base, 16)] = jnp.where(
                lax.iota(jnp.int32, 16) == lane, val, v)

        def batch(b, carry):
            astart = jnp.minimum(lo8 + b * _SSUB, p - _SSUB)
            bstart = lo8 + b * _SSUB
            maxlo = jnp.maximum(lo, bstart)
            pltpu.sync_copy(order_hbm.at[pl.ds(astart, _SSUB)],
                            ord_v.at[pl.ds(0, _SSUB)])
            pltpu.sync_copy(sdst_hbm.at[pl.ds(astart, _SSUB)],
                            sdst_v.at[pl.ds(0, _SSUB)])
            for k in range(_SSUB // 16):
                posv = astart + k * 16 + lax.iota(jnp.int32, 16)
                okv = (posv >= maxlo) & (posv < hi)
                ov = ord_v[pl.ds(k * 16, 16)]
                ord_v[pl.ds(k * 16, 16)] = jnp.where(okv, ov, p - 1)
            cp = pltpu.async_copy(vals_hbm.at[ord_v.at[pl.ds(0, _SSUB)]],
                                  gbuf, sem)
            cp.wait()

            def row(j, c2):
                cur, oc = c2
                d_r = _read_i32(sdst_v, j)
                pos = astart + j
                valid = (pos >= maxlo) & (pos < hi)
                flush = valid & (d_r != cur)
                do_stage = flush & (cur >= 0)

                @pl.when(do_stage)
                def _stage():
                    slot = oc % oslots
                    orow = obuf.at[slot]
                    for k in range(nvec):
                        orow[pl.ds(k * 16, 16)] = abuf[pl.ds(k * 16, 16)]
                    set_dlist(slot, cur)

                oc2 = jnp.where(do_stage, oc + 1, oc)

                @pl.when(do_stage & (oc2 % oslots == 0))
                def _drain():
                    flush_obuf()
                    reset_dlist()

                grow = gbuf.at[j]

                @pl.when(flush)
                def _restart():
                    for k in range(nvec):
                        abuf[pl.ds(k * 16, 16)] = grow[pl.ds(k * 16, 16)]

                @pl.when(jnp.logical_not(flush))
                def _accum():
                    for k in range(nvec):
                        abuf[pl.ds(k * 16, 16)] = (
                            abuf[pl.ds(k * 16, 16)]
                            + grow[pl.ds(k * 16, 16)])

                cur2 = jnp.where(flush, d_r, cur)
                return (cur2, oc2)

            return lax.fori_loop(0, _SSUB, row, carry)

        init = (jnp.int32(-1), jnp.int32(0))
        reset_dlist()
        carry = lax.fori_loop(0, nb, batch, init)
        cur, oc = carry

        @pl.when(cur >= 0)
        def _final_stage():
            slot = oc % oslots
            orow = obuf.at[slot]
            for k in range(nvec):
                orow[pl.ds(k * 16, 16)] = abuf[pl.ds(k * 16, 16)]
            set_dlist(slot, cur)

        flush_obuf()

    zeros = jnp.zeros((zrows, d), jnp.float32)
    f = pl.kernel(
        body,
        out_type=jax.ShapeDtypeStruct((m_pad + 8, d), jnp.float32),
        mesh=mesh,
        scratch_types=[
            pltpu.VMEM((64,), jnp.int32),
            pltpu.VMEM((_SSUB + 16,), jnp.int32),
            pltpu.VMEM((_SSUB + 16,), jnp.int32),
            pltpu.VMEM((oslots,), jnp.int32),
            pltpu.VMEM((_SSUB, d), jnp.float32),
            pltpu.VMEM((oslots, d), jnp.float32),
            pltpu.VMEM((zrows, d), jnp.float32),
            pltpu.VMEM((d,), jnp.float32),
            pltpu.SemaphoreType.DMA,
        ],
    )
    return f(vals, order, sdst, tbounds, zeros)


def _relmm_kernel(blk_rel_ref, g_ref, w_ref, rw_ref, o_ref):
    acc = jnp.dot(g_ref[...], w_ref[0], preferred_element_type=jnp.float32)
    o_ref[...] = acc * rw_ref[...]


def _rel_matmul(g, wstack, row_w, blk_rel):
    p, dk = g.shape
    r, dk2, dn = wstack.shape
    assert dk == dk2 and p % BLK == 0
    return pl.pallas_call(
        _relmm_kernel,
        grid_spec=pltpu.PrefetchScalarGridSpec(
            num_scalar_prefetch=1,
            grid=(p // BLK,),
            in_specs=[
                pl.BlockSpec((BLK, dk), lambda i, br: (i, 0)),
                pl.BlockSpec((1, dk, dn), lambda i, br: (br[i], 0, 0)),
                pl.BlockSpec((BLK, 1), lambda i, br: (i, 0)),
            ],
            out_specs=pl.BlockSpec((BLK, dn), lambda i, br: (i, 0)),
        ),
        out_shape=jax.ShapeDtypeStruct((p, dn), jnp.float32),
    )(blk_rel, g, wstack, row_w)


def _mm_kernel(a_ref, b_ref, bias_ref, o_ref, *, relu):
    acc = jnp.dot(a_ref[...], b_ref[...], preferred_element_type=jnp.float32)
    acc = acc + bias_ref[...]
    if relu:
        acc = jnp.maximum(acc, 0.0)
    o_ref[...] = acc


def _matmul(a, b, bias, relu=False, bm=1024):
    m, k = a.shape
    k2, n = b.shape
    assert k == k2
    mp = _round_up(m, bm)
    kp = _round_up(k, 128)
    np_ = _round_up(n, 128)
    a = jnp.pad(a, ((0, mp - m), (0, kp - k)))
    b = jnp.pad(b, ((0, kp - k), (0, np_ - n)))
    bias = jnp.pad(bias, ((0, np_ - n),)).reshape(1, np_)
    out = pl.pallas_call(
        functools.partial(_mm_kernel, relu=relu),
        grid=(mp // bm,),
        in_specs=[
            pl.BlockSpec((bm, kp), lambda i: (i, 0)),
            pl.BlockSpec((kp, np_), lambda i: (0, 0)),
            pl.BlockSpec((1, np_), lambda i: (0, 0)),
        ],
        out_specs=pl.BlockSpec((bm, np_), lambda i: (i, 0)),
        out_shape=jax.ShapeDtypeStruct((mp, np_), jnp.float32),
    )(a, b, bias)
    return out[:m, :n]


def _sorted_rel_plan(rel, num_rel, n_edges):
    p = _round_up((_round_up(n_edges, BLK) // BLK + num_rel) * BLK,
                  _NW * _SUB)
    perm = jnp.argsort(rel)
    counts = jnp.bincount(rel, length=num_rel)
    off = jnp.concatenate([jnp.zeros((1,), jnp.int32),
                           jnp.cumsum(counts).astype(jnp.int32)])
    blocks_r = (counts + BLK - 1) // BLK
    pad_off = BLK * jnp.concatenate([jnp.zeros((1,), jnp.int32),
                                     jnp.cumsum(blocks_r).astype(jnp.int32)])
    j = jnp.arange(p, dtype=jnp.int32)
    r_j = jnp.clip(jnp.searchsorted(pad_off, j, side="right") - 1,
                   0, num_rel - 1).astype(jnp.int32)
    k = j - pad_off[r_j]
    valid = k < counts[r_j]
    e_map = perm[jnp.clip(off[r_j] + k, 0, n_edges - 1)]
    e_map = jnp.where(valid, e_map, 0)
    blk_rel = jnp.clip(
        jnp.searchsorted(pad_off, jnp.arange(p // BLK, dtype=jnp.int32) * BLK,
                         side="right") - 1, 0, num_rel - 1).astype(jnp.int32)
    return e_map, valid, blk_rel, p


def _dst_plan(dst_pad, m):
    m_pad = _round_up(m, _NW * 8)
    rows_pt = m_pad // _NW
    order = jnp.argsort(dst_pad).astype(jnp.int32)
    sdst = dst_pad[order]
    tb = jnp.searchsorted(
        sdst, jnp.arange(_NW + 1, dtype=jnp.int32) * rows_pt
    ).astype(jnp.int32)
    tb = jnp.pad(tb, (0, 64 - (_NW + 1)))
    return order, sdst, tb, m_pad


def _pad_cols(x, dg):
    return jnp.pad(x, ((0, 0), (0, dg - x.shape[1])))


def _gcols(d):
    return _round_up(d, 128)


def _split_w(linw, num_rel, d_in, dk_pad, dn_pad):
    d_out = linw.shape[1]
    w = linw.reshape(num_rel, d_in, d_out)
    return jnp.pad(w, ((0, 0), (0, dk_pad - d_in), (0, dn_pad - d_out)))


def _bn(x, g, b):
    m = jnp.mean(x, axis=0)
    v = jnp.var(x, axis=0)
    return (x - m) / jnp.sqrt(v + EPS) * g + b


def _msg_aggregate(x, gather_idx, linw, num_rel, w_pad, blk_rel, dplan, m_out):
    d_in = x.shape[1]
    d_out = linw.shape[1]
    dk = _gcols(d_in)
    dn_pad = _round_up(d_out, 128)
    g = _sc_gather(_pad_cols(x, dk), gather_idx)
    wstack = _split_w(linw, num_rel, d_in, dk, dn_pad)
    mm = _rel_matmul(g, wstack, w_pad, blk_rel)
    order, sdst, tbounds, m_pad = dplan
    out = _sc_scatter(mm, order, sdst, tbounds, m_pad)
    return out[:m_out, :d_out]


def kernel(node_feature, edge_index, edge_relation, edge_feature, edge_weight,
           line_edge_index, line_edge_relation, line_edge_weight, node2graph,
           params):
    e_map_n, valid_n, blk_rel_n, p_n = _sorted_rel_plan(edge_relation,
                                                        NUM_REL, E)
    e_map_l, valid_l, blk_rel_l, p_l = _sorted_rel_plan(line_edge_relation,
                                                        NUM_ANGLE, E2)
    dst_n = jnp.where(valid_n, edge_index[1][e_map_n], 0).astype(jnp.int32)
    dst_l = jnp.where(valid_l, line_edge_index[1][e_map_l], 0).astype(jnp.int32)
    w_n = jnp.where(valid_n, edge_weight[e_map_n], 0.0)[:, None]
    w_l = jnp.where(valid_l, line_edge_weight[e_map_l], 0.0)[:, None]
    src_n = jnp.where(valid_n, edge_index[0][e_map_n], 0).astype(jnp.int32)
    src_l = jnp.where(valid_l, line_edge_index[0][e_map_l], 0).astype(jnp.int32)
    upd_gidx = jnp.where(valid_n, e_map_n, 0).astype(jnp.int32)
    dplan_n = _dst_plan(dst_n, N)
    dplan_l = _dst_plan(dst_l, E)

    hiddens = []
    layer_input = node_feature
    edge_input = edge_feature
    for i in range(3):
        pn = params["node"][i]
        pe = params["edge"][i]
        s = _msg_aggregate(layer_input, src_n, pn["linW"], NUM_REL, w_n,
                           blk_rel_n, dplan_n, N)
        y = s + pn["linb"] + _matmul(layer_input, pn["slW"], pn["slb"])
        hidden = jax.nn.relu(_bn(y, pn["bng"], pn["bnb"]))
        if hidden.shape == layer_input.shape:
            hidden = hidden + layer_input
        s2 = _msg_aggregate(edge_input, src_l, pe["linW"], NUM_ANGLE, w_l,
                            blk_rel_l, dplan_l, E)
        y2 = s2 + pe["linb"] + _matmul(edge_input, pe["slW"], pe["slb"])
        edge_hidden = jax.nn.relu(_bn(y2, pe["bng"], pe["bnb"]))
        upd = _msg_aggregate(edge_hidden, upd_gidx, pn["linW"], NUM_REL, w_n,
                             blk_rel_n, dplan_n, N)
        upd = jax.nn.relu(upd + pn["linb"])
        hidden = hidden + upd
        edge_input = edge_hidden
        hidden = _bn(hidden, params["bn"][i]["g"], params["bn"][i]["b"])
        hiddens.append(hidden)
        layer_input = hidden
    node_feat = jnp.concatenate(hiddens, axis=-1)
    graph_feat = jax.ops.segment_sum(node_feat, node2graph,
                                     num_segments=NUM_GRAPHS)
    return graph_feat, node_feat


# speedup vs baseline: 2.0778x; 2.0778x over previous
import functools

import jax
import jax.numpy as jnp
from jax import lax
from jax.experimental import pallas as pl
from jax.experimental.pallas import tpu as pltpu
from jax.experimental.pallas import tpu_sc as plsc

N = 10000
E = 40000
E2 = 120000
NUM_REL = 7
NUM_ANGLE = 8
NUM_GRAPHS = 32
EPS = 1e-5
BLK = 512

_NC = 2
_NS = 16
_NW = _NC * _NS
_SUB = 64
_SSUB = 128
_CROWS = 2048


def _round_up(x, m):
    return ((x + m - 1) // m) * m


def _sc_gather(table, idx):
    t, d = table.shape
    p = idx.shape[0]
    assert p % (_NW * _SUB) == 0 and (d * 4) % 64 == 0
    per_w = p // _NW
    nsub = per_w // _SUB
    mesh = plsc.VectorSubcoreMesh(core_axis_name="c", subcore_axis_name="s")

    def body(table_hbm, idx_hbm, out_hbm, idx_v, buf0, buf1, buf2,
             g0, g1, g2, s0, s1, s2):
        wid = lax.axis_index("s") * _NC + lax.axis_index("c")
        base = wid * per_w
        pltpu.sync_copy(idx_hbm.at[pl.ds(base, per_w)], idx_v)
        bufs = (buf0, buf1, buf2)
        gsem = (g0, g1, g2)
        ssem = (s0, s1, s2)
        gath = [None, None, None]
        stor = [None, None, None]

        def start_gather(j):
            s = j % 3
            gath[s] = pltpu.async_copy(
                table_hbm.at[idx_v.at[pl.ds(j * _SUB, _SUB)]],
                bufs[s], gsem[s])

        for j in range(min(2, nsub)):
            start_gather(j)
        for j in range(nsub):
            s = j % 3
            sn = (j + 2) % 3
            if j + 2 < nsub:
                if stor[sn] is not None:
                    stor[sn].wait()
                    stor[sn] = None
                start_gather(j + 2)
            gath[s].wait()
            if stor[s] is not None:
                stor[s].wait()
            stor[s] = pltpu.async_copy(
                bufs[s], out_hbm.at[pl.ds(base + j * _SUB, _SUB)], ssem[s])
        for s in range(3):
            if stor[s] is not None:
                stor[s].wait()

    f = pl.kernel(
        body,
        out_type=jax.ShapeDtypeStruct((p, d), jnp.float32),
        mesh=mesh,
        scratch_types=[
            pltpu.VMEM((per_w,), jnp.int32),
            pltpu.VMEM((_SUB, d), jnp.float32),
            pltpu.VMEM((_SUB, d), jnp.float32),
            pltpu.VMEM((_SUB, d), jnp.float32),
            pltpu.SemaphoreType.DMA,
            pltpu.SemaphoreType.DMA,
            pltpu.SemaphoreType.DMA,
            pltpu.SemaphoreType.DMA,
            pltpu.SemaphoreType.DMA,
            pltpu.SemaphoreType.DMA,
        ],
    )
    return f(table, idx)


def _sc_scatter(vals, order, sdst, tbounds, m_pad):
    p, d = vals.shape
    assert (d * 4) % 64 == 0 and p % 8 == 0 and m_pad % (_NW * 8) == 0
    rows_pt = m_pad // _NW
    nvec = d // 16
    zrows = 32
    oslots = 64
    mesh = plsc.VectorSubcoreMesh(core_axis_name="c", subcore_axis_name="s")

    def _read_i32(ref, idx):
        return ref[pl.ds(idx, 16)][0]

    def body(vals_hbm, order_hbm, sdst_hbm, tb_hbm, zeros_hbm, out_hbm,
             tb_v, ord_v, sdst_v, dlist_v, gbuf, obuf, zbuf, sem):
        cid = lax.axis_index("c")
        sid = lax.axis_index("s")
        wid = sid * _NC + cid
        pltpu.sync_copy(tb_hbm, tb_v)
        pltpu.sync_copy(zeros_hbm, zbuf)
        obase = wid * rows_pt
        for h in range(rows_pt // zrows):
            pltpu.sync_copy(zbuf, out_hbm.at[pl.ds(obase + h * zrows, zrows)])
        lo = _read_i32(tb_v, wid)
        hi = _read_i32(tb_v, wid + 1)
        lo8 = (lo // 8) * 8
        nb = (hi - lo8 + _SSUB - 1) // _SSUB

        def flush_obuf():
            pltpu.sync_copy(obuf, out_hbm.at[dlist_v])

        def reset_dlist():
            for k in range(oslots // 16):
                dlist_v[pl.ds(k * 16, 16)] = jnp.broadcast_to(
                    jnp.int32(m_pad), (16,))

        def set_dlist(slot, val):
            base = (slot // 16) * 16
            lane = slot % 16
            v = dlist_v[pl.ds(base, 16)]
            dlist_v[pl.ds(base, 16)] = jnp.where(
                lax.iota(jnp.int32, 16) == lane, val, v)

        def batch(b, carry):
            astart = jnp.minimum(lo8 + b * _SSUB, p - _SSUB)
            bstart = lo8 + b * _SSUB
            maxlo = jnp.maximum(lo, bstart)
            pltpu.sync_copy(order_hbm.at[pl.ds(astart, _SSUB)],
                            ord_v.at[pl.ds(0, _SSUB)])
            pltpu.sync_copy(sdst_hbm.at[pl.ds(astart, _SSUB)],
                            sdst_v.at[pl.ds(0, _SSUB)])
            for k in range(_SSUB // 16):
                posv = astart + k * 16 + lax.iota(jnp.int32, 16)
                okv = (posv >= maxlo) & (posv < hi)
                ov = ord_v[pl.ds(k * 16, 16)]
                ord_v[pl.ds(k * 16, 16)] = jnp.where(okv, ov, p - 1)
            cp = pltpu.async_copy(vals_hbm.at[ord_v.at[pl.ds(0, _SSUB)]],
                                  gbuf, sem)
            cp.wait()

            def row(j, c2):
                cur = c2[0]
                oc = c2[1]
                acc = c2[2:]
                d_r = _read_i32(sdst_v, j)
                pos = astart + j
                valid = (pos >= maxlo) & (pos < hi)
                flush = valid & (d_r != cur)
                do_stage = flush & (cur >= 0)

                @pl.when(do_stage)
                def _stage():
                    slot = oc % oslots
                    orow = obuf.at[slot]
                    for k in range(nvec):
                        orow[pl.ds(k * 16, 16)] = acc[k]
                    set_dlist(slot, cur)

                oc2 = jnp.where(do_stage, oc + 1, oc)

                @pl.when(do_stage & (oc2 % oslots == 0))
                def _drain():
                    flush_obuf()
                    reset_dlist()

                grow = gbuf.at[j]
                newacc = []
                for k in range(nvec):
                    v = grow[pl.ds(k * 16, 16)]
                    a = acc[k]
                    newacc.append(jnp.where(flush, v, a + v))
                cur2 = jnp.where(flush, d_r, cur)
                return (cur2, oc2, *newacc)

            return lax.fori_loop(0, _SSUB, row, carry)

        init = (jnp.int32(-1), jnp.int32(0),
                *(jnp.zeros((16,), jnp.float32),) * nvec)
        reset_dlist()
        carry = lax.fori_loop(0, nb, batch, init)
        cur = carry[0]
        oc = carry[1]
        acc = carry[2:]

        @pl.when(cur >= 0)
        def _final_stage():
            slot = oc % oslots
            orow = obuf.at[slot]
            for k in range(nvec):
                orow[pl.ds(k * 16, 16)] = acc[k]
            set_dlist(slot, cur)

        flush_obuf()

    zeros = jnp.zeros((zrows, d), jnp.float32)
    f = pl.kernel(
        body,
        out_type=jax.ShapeDtypeStruct((m_pad + 8, d), jnp.float32),
        mesh=mesh,
        scratch_types=[
            pltpu.VMEM((64,), jnp.int32),
            pltpu.VMEM((_SSUB + 16,), jnp.int32),
            pltpu.VMEM((_SSUB + 16,), jnp.int32),
            pltpu.VMEM((oslots,), jnp.int32),
            pltpu.VMEM((_SSUB, d), jnp.float32),
            pltpu.VMEM((oslots, d), jnp.float32),
            pltpu.VMEM((zrows, d), jnp.float32),
            pltpu.SemaphoreType.DMA,
        ],
    )
    return f(vals, order, sdst, tbounds, zeros)


def _relmm_kernel(blk_rel_ref, g_ref, w_ref, rw_ref, o_ref):
    acc = jnp.dot(g_ref[...], w_ref[0], preferred_element_type=jnp.float32)
    o_ref[...] = acc * rw_ref[...]


def _rel_matmul(g, wstack, row_w, blk_rel):
    p, dk = g.shape
    r, dk2, dn = wstack.shape
    assert dk == dk2 and p % BLK == 0
    return pl.pallas_call(
        _relmm_kernel,
        grid_spec=pltpu.PrefetchScalarGridSpec(
            num_scalar_prefetch=1,
            grid=(p // BLK,),
            in_specs=[
                pl.BlockSpec((BLK, dk), lambda i, br: (i, 0)),
                pl.BlockSpec((1, dk, dn), lambda i, br: (br[i], 0, 0)),
                pl.BlockSpec((BLK, 1), lambda i, br: (i, 0)),
            ],
            out_specs=pl.BlockSpec((BLK, dn), lambda i, br: (i, 0)),
        ),
        out_shape=jax.ShapeDtypeStruct((p, dn), jnp.float32),
    )(blk_rel, g, wstack, row_w)


def _mm_kernel(a_ref, b_ref, bias_ref, o_ref, *, relu):
    acc = jnp.dot(a_ref[...], b_ref[...], preferred_element_type=jnp.float32)
    acc = acc + bias_ref[...]
    if relu:
        acc = jnp.maximum(acc, 0.0)
    o_ref[...] = acc


def _matmul(a, b, bias, relu=False, bm=1024):
    m, k = a.shape
    k2, n = b.shape
    assert k == k2
    mp = _round_up(m, bm)
    kp = _round_up(k, 128)
    np_ = _round_up(n, 128)
    a = jnp.pad(a, ((0, mp - m), (0, kp - k)))
    b = jnp.pad(b, ((0, kp - k), (0, np_ - n)))
    bias = jnp.pad(bias, ((0, np_ - n),)).reshape(1, np_)
    out = pl.pallas_call(
        functools.partial(_mm_kernel, relu=relu),
        grid=(mp // bm,),
        in_specs=[
            pl.BlockSpec((bm, kp), lambda i: (i, 0)),
            pl.BlockSpec((kp, np_), lambda i: (0, 0)),
            pl.BlockSpec((1, np_), lambda i: (0, 0)),
        ],
        out_specs=pl.BlockSpec((bm, np_), lambda i: (i, 0)),
        out_shape=jax.ShapeDtypeStruct((mp, np_), jnp.float32),
    )(a, b, bias)
    return out[:m, :n]


def _readout_kernel(n2g_ref, x_ref, o_ref, *, bm):
    i = pl.program_id(0)
    oh = (n2g_ref[0] == jax.lax.broadcasted_iota(jnp.int32,
                                                 (NUM_GRAPHS, bm), 0))
    part = jnp.dot(oh.astype(jnp.float32), x_ref[...],
                   preferred_element_type=jnp.float32)

    @pl.when(i == 0)
    def _init():
        o_ref[...] = jnp.zeros_like(o_ref)

    o_ref[...] += part


def _readout(x, n2g):
    n, d = x.shape
    bm = 2048
    npd = _round_up(n, bm)
    x = jnp.pad(x, ((0, npd - n), (0, 0)))
    n2g = jnp.pad(n2g, (0, npd - n), constant_values=NUM_GRAPHS)
    n2g = n2g.reshape(npd // bm, 1, bm)
    return pl.pallas_call(
        functools.partial(_readout_kernel, bm=bm),
        grid=(npd // bm,),
        in_specs=[
            pl.BlockSpec((1, 1, bm), lambda i: (i, 0, 0)),
            pl.BlockSpec((bm, d), lambda i: (i, 0)),
        ],
        out_specs=pl.BlockSpec((NUM_GRAPHS, d), lambda i: (0, 0)),
        out_shape=jax.ShapeDtypeStruct((NUM_GRAPHS, d), jnp.float32),
    )(n2g, x)


def _sorted_rel_plan(rel, num_rel, n_edges):
    p = _round_up((_round_up(n_edges, BLK) // BLK + num_rel) * BLK,
                  _NW * _SUB)
    perm = jnp.argsort(rel)
    counts = jnp.bincount(rel, length=num_rel)
    off = jnp.concatenate([jnp.zeros((1,), jnp.int32),
                           jnp.cumsum(counts).astype(jnp.int32)])
    blocks_r = (counts + BLK - 1) // BLK
    pad_off = BLK * jnp.concatenate([jnp.zeros((1,), jnp.int32),
                                     jnp.cumsum(blocks_r).astype(jnp.int32)])
    j = jnp.arange(p, dtype=jnp.int32)
    r_j = jnp.clip(jnp.searchsorted(pad_off, j, side="right") - 1,
                   0, num_rel - 1).astype(jnp.int32)
    k = j - pad_off[r_j]
    valid = k < counts[r_j]
    e_map = perm[jnp.clip(off[r_j] + k, 0, n_edges - 1)]
    e_map = jnp.where(valid, e_map, 0)
    blk_rel = jnp.clip(
        jnp.searchsorted(pad_off, jnp.arange(p // BLK, dtype=jnp.int32) * BLK,
                         side="right") - 1, 0, num_rel - 1).astype(jnp.int32)
    return e_map, valid, blk_rel, p


def _dst_plan(dst_pad, m):
    m_pad = _round_up(m, _NW * 8)
    rows_pt = m_pad // _NW
    order = jnp.argsort(dst_pad).astype(jnp.int32)
    sdst = dst_pad[order]
    tb = jnp.searchsorted(
        sdst, jnp.arange(_NW + 1, dtype=jnp.int32) * rows_pt
    ).astype(jnp.int32)
    tb = jnp.pad(tb, (0, 64 - (_NW + 1)))
    return order, sdst, tb, m_pad


def _pad_cols(x, dg):
    return jnp.pad(x, ((0, 0), (0, dg - x.shape[1])))


def _gcols(d):
    return _round_up(d, 128)


def _split_w(linw, num_rel, d_in, dk_pad, dn_pad):
    d_out = linw.shape[1]
    w = linw.reshape(num_rel, d_in, d_out)
    return jnp.pad(w, ((0, 0), (0, dk_pad - d_in), (0, dn_pad - d_out)))


def _bn(x, g, b):
    m = jnp.mean(x, axis=0)
    v = jnp.var(x, axis=0)
    return (x - m) / jnp.sqrt(v + EPS) * g + b


def _msg_aggregate(x, gather_idx, linw, num_rel, w_pad, blk_rel, dplan, m_out):
    d_in = x.shape[1]
    d_out = linw.shape[1]
    dk = _gcols(d_in)
    dn_pad = _round_up(d_out, 128)
    g = _sc_gather(_pad_cols(x, dk), gather_idx)
    wstack = _split_w(linw, num_rel, d_in, dk, dn_pad)
    mm = _rel_matmul(g, wstack, w_pad, blk_rel)
    order, sdst, tbounds, m_pad = dplan
    out = _sc_scatter(mm, order, sdst, tbounds, m_pad)
    return out[:m_out, :d_out]


def kernel(node_feature, edge_index, edge_relation, edge_feature, edge_weight,
           line_edge_index, line_edge_relation, line_edge_weight, node2graph,
           params):
    e_map_n, valid_n, blk_rel_n, p_n = _sorted_rel_plan(edge_relation,
                                                        NUM_REL, E)
    e_map_l, valid_l, blk_rel_l, p_l = _sorted_rel_plan(line_edge_relation,
                                                        NUM_ANGLE, E2)
    dst_n = jnp.where(valid_n, edge_index[1][e_map_n], 0).astype(jnp.int32)
    dst_l = jnp.where(valid_l, line_edge_index[1][e_map_l], 0).astype(jnp.int32)
    w_n = jnp.where(valid_n, edge_weight[e_map_n], 0.0)[:, None]
    w_l = jnp.where(valid_l, line_edge_weight[e_map_l], 0.0)[:, None]
    src_n = jnp.where(valid_n, edge_index[0][e_map_n], 0).astype(jnp.int32)
    src_l = jnp.where(valid_l, line_edge_index[0][e_map_l], 0).astype(jnp.int32)
    upd_gidx = jnp.where(valid_n, e_map_n, 0).astype(jnp.int32)
    dplan_n = _dst_plan(dst_n, N)
    dplan_l = _dst_plan(dst_l, E)

    hiddens = []
    layer_input = node_feature
    edge_input = edge_feature
    for i in range(3):
        pn = params["node"][i]
        pe = params["edge"][i]
        s = _msg_aggregate(layer_input, src_n, pn["linW"], NUM_REL, w_n,
                           blk_rel_n, dplan_n, N)
        y = s + pn["linb"] + _matmul(layer_input, pn["slW"], pn["slb"])
        hidden = jax.nn.relu(_bn(y, pn["bng"], pn["bnb"]))
        if hidden.shape == layer_input.shape:
            hidden = hidden + layer_input
        s2 = _msg_aggregate(edge_input, src_l, pe["linW"], NUM_ANGLE, w_l,
                            blk_rel_l, dplan_l, E)
        y2 = s2 + pe["linb"] + _matmul(edge_input, pe["slW"], pe["slb"])
        edge_hidden = jax.nn.relu(_bn(y2, pe["bng"], pe["bnb"]))
        upd = _msg_aggregate(edge_hidden, upd_gidx, pn["linW"], NUM_REL, w_n,
                             blk_rel_n, dplan_n, N)
        upd = jax.nn.relu(upd + pn["linb"])
        hidden = hidden + upd
        edge_input = edge_hidden
        hidden = _bn(hidden, params["bn"][i]["g"], params["bn"][i]["b"])
        hiddens.append(hidden)
        layer_input = hidden
    node_feat = jnp.concatenate(hiddens, axis=-1)
    graph_feat = _readout(node_feat, node2graph)
    return graph_feat, node_feat
